# Initial kernel scaffold; baseline (speedup 1.0000x reference)
import jax
import jax.numpy as jnp
from jax.experimental import pallas as pl


def kernel(pe, time_idx):
    raise NotImplementedError("write your pallas kernel here")


# speedup vs baseline: 5.1064x; 5.1064x over previous
import jax
import jax.numpy as jnp
from jax import lax
from jax.experimental import pallas as pl
from jax.experimental.pallas import tpu as pltpu
from jax.experimental.pallas import tpu_sc as plsc

MAX_LEN = 4000
D = 128
BATCH = 4096
HIST = 200
B_TOTAL = BATCH * HIST
NC = 2
NS = 16
NW = NC * NS
B_PER_W = B_TOTAL // NW
CHUNK = 128
N_CHUNKS = B_PER_W // CHUNK


def _gather_body(table_hbm, idx_hbm, out_hbm, idx_v, rows_v, sem):
    wid = lax.axis_index("s") * NC + lax.axis_index("c")
    base = wid * B_PER_W

    def chunk(j, carry):
        off = base + j * CHUNK
        pltpu.sync_copy(idx_hbm.at[pl.ds(off, CHUNK)], idx_v)
        pltpu.async_copy(table_hbm.at[idx_v], rows_v, sem).wait()
        pltpu.sync_copy(rows_v, out_hbm.at[pl.ds(off, CHUNK)])
        return carry

    lax.fori_loop(0, N_CHUNKS, chunk, 0)


def kernel(pe, time_idx):
    table = pe.reshape(MAX_LEN, D)
    idx = time_idx.reshape(-1).astype(jnp.int32)
    mesh = plsc.VectorSubcoreMesh(core_axis_name="c", subcore_axis_name="s")
    out = pl.kernel(
        _gather_body,
        out_type=jax.ShapeDtypeStruct((B_TOTAL, D), jnp.float32),
        mesh=mesh,
        scratch_types=[
            pltpu.VMEM((CHUNK,), jnp.int32),
            pltpu.VMEM((CHUNK, D), jnp.float32),
            pltpu.SemaphoreType.DMA,
        ],
    )(table, idx)
    return out.reshape(BATCH, HIST, D)


# speedup vs baseline: 8.0782x; 1.5820x over previous
import jax
import jax.numpy as jnp
from jax import lax
from jax.experimental import pallas as pl
from jax.experimental.pallas import tpu as pltpu
from jax.experimental.pallas import tpu_sc as plsc

MAX_LEN = 4000
D = 128
BATCH = 4096
HIST = 200
B_TOTAL = BATCH * HIST
NC = 2
NS = 16
NW = NC * NS
B_PER_W = B_TOTAL // NW
CHUNK = 128
N_CHUNKS = B_PER_W // CHUNK
NBUF = 4
GROUP = NBUF * CHUNK
N_GROUPS = N_CHUNKS // NBUF


def _gather_body(table_hbm, idx_hbm, out_hbm, idx_all, rows, *sems):
    gsems, wsems = sems[:NBUF], sems[NBUF:]
    wid = lax.axis_index("s") * NC + lax.axis_index("c")
    base = wid * B_PER_W

    pltpu.sync_copy(idx_hbm.at[wid], idx_all)

    def group(g, carry):
        goff = base + g * GROUP
        descs = []
        for b in range(NBUF):
            @pl.when(g > 0)
            def _wait_prev_write(b=b):
                pltpu.make_async_copy(
                    rows.at[b], out_hbm.at[pl.ds(goff + b * CHUNK, CHUNK)], wsems[b]
                ).wait()

            d = pltpu.make_async_copy(
                table_hbm.at[idx_all.at[g * NBUF + b]], rows.at[b], gsems[b]
            )
            d.start()
            descs.append(d)
        for b in range(NBUF):
            descs[b].wait()
            pltpu.async_copy(
                rows.at[b], out_hbm.at[pl.ds(goff + b * CHUNK, CHUNK)], wsems[b]
            )
        return carry

    lax.fori_loop(0, N_GROUPS, group, 0)

    for b in range(NBUF):
        pltpu.make_async_copy(
            rows.at[b], out_hbm.at[pl.ds(base + b * CHUNK, CHUNK)], wsems[b]
        ).wait()


def kernel(pe, time_idx):
    table = pe.reshape(MAX_LEN, D)
    idx = time_idx.reshape(NW, N_CHUNKS, CHUNK).astype(jnp.int32)
    mesh = plsc.VectorSubcoreMesh(core_axis_name="c", subcore_axis_name="s")
    out = pl.kernel(
        _gather_body,
        out_type=jax.ShapeDtypeStruct((B_TOTAL, D), jnp.float32),
        mesh=mesh,
        scratch_types=[
            pltpu.VMEM((N_CHUNKS, CHUNK), jnp.int32),
            pltpu.VMEM((NBUF, CHUNK, D), jnp.float32),
        ]
        + [pltpu.SemaphoreType.DMA] * (2 * NBUF),
    )(table, idx)
    return out.reshape(BATCH, HIST, D)


# speedup vs baseline: 15.5593x; 1.9261x over previous
import jax
import jax.numpy as jnp
from jax import lax
from jax.experimental import pallas as pl
from jax.experimental.pallas import tpu as pltpu
from jax.experimental.pallas import tpu_sc as plsc

MAX_LEN = 4000
D = 128
BATCH = 4096
HIST = 200
B_TOTAL = BATCH * HIST
NC = 2
NS = 16
NW = NC * NS
B_PER_W = B_TOTAL // NW
CHUNK = 128
N_CHUNKS = B_PER_W // CHUNK
NBUF = 4
GROUP = NBUF * CHUNK
N_GROUPS = N_CHUNKS // NBUF


def _gather_body(table_hbm, idx_hbm, out_hbm, table_sp, idx_all, rows, *sems):
    gsems, wsems = sems[:NBUF], sems[NBUF:]
    sid = lax.axis_index("s")
    wid = sid * NC + lax.axis_index("c")
    base = wid * B_PER_W

    @pl.when(sid == 0)
    def _stage_table():
        pltpu.sync_copy(table_hbm, table_sp)

    pltpu.sync_copy(idx_hbm.at[wid], idx_all)
    plsc.subcore_barrier()

    def group(g, carry):
        goff = base + g * GROUP
        descs = []
        for b in range(NBUF):
            @pl.when(g > 0)
            def _wait_prev_write(b=b):
                pltpu.make_async_copy(
                    rows.at[b], out_hbm.at[pl.ds(goff + b * CHUNK, CHUNK)], wsems[b]
                ).wait()

            d = pltpu.make_async_copy(
                table_sp.at[idx_all.at[g * NBUF + b]], rows.at[b], gsems[b]
            )
            d.start()
            descs.append(d)
        for b in range(NBUF):
            descs[b].wait()
            pltpu.async_copy(
                rows.at[b], out_hbm.at[pl.ds(goff + b * CHUNK, CHUNK)], wsems[b]
            )
        return carry

    lax.fori_loop(0, N_GROUPS, group, 0)

    for b in range(NBUF):
        pltpu.make_async_copy(
            rows.at[b], out_hbm.at[pl.ds(base + b * CHUNK, CHUNK)], wsems[b]
        ).wait()


def kernel(pe, time_idx):
    table = pe.reshape(MAX_LEN, D)
    idx = time_idx.reshape(NW, N_CHUNKS, CHUNK).astype(jnp.int32)
    mesh = plsc.VectorSubcoreMesh(core_axis_name="c", subcore_axis_name="s")
    out = pl.kernel(
        _gather_body,
        out_type=jax.ShapeDtypeStruct((B_TOTAL, D), jnp.float32),
        mesh=mesh,
        scratch_types=[
            pltpu.VMEM_SHARED((MAX_LEN, D), jnp.float32),
            pltpu.VMEM((N_CHUNKS, CHUNK), jnp.int32),
            pltpu.VMEM((NBUF, CHUNK, D), jnp.float32),
        ]
        + [pltpu.SemaphoreType.DMA] * (2 * NBUF),
    )(table, idx)
    return out.reshape(BATCH, HIST, D)
